# baseline (speedup 1.0000x reference)
import functools

import jax
import jax.numpy as jnp
from jax import lax
from jax.experimental import pallas as pl
from jax.experimental.pallas import tpu as pltpu
from jax.experimental.pallas import tpu_sc as plsc

_MAX_T = 100000
_EMB_DIM = 16
_BATCH = 16384

_NC = 2
_NS = 16
_NW = _NC * _NS
_B_PER_W = _BATCH // _NW

_mesh = plsc.VectorSubcoreMesh(core_axis_name="c", subcore_axis_name="s")


@functools.partial(
    pl.kernel,
    mesh=_mesh,
    out_type=jax.ShapeDtypeStruct((_BATCH, _EMB_DIM), jnp.float32),
    scratch_types=[
        pltpu.VMEM((_B_PER_W,), jnp.int32),
        pltpu.SemaphoreType.DMA,
    ],
)
def _gather_kernel(table_hbm, idx_hbm, out_hbm, idx_v, sem):
    wid = lax.axis_index("s") * _NC + lax.axis_index("c")
    base = wid * _B_PER_W
    pltpu.sync_copy(idx_hbm.at[pl.ds(base, _B_PER_W)], idx_v)

    def body(g, _):
        vec = idx_v[pl.ds(g * 16, 16)]
        for lane in range(16):
            row = vec[lane]
            pltpu.async_copy(
                table_hbm.at[pl.ds(row, 1), :],
                out_hbm.at[pl.ds(base + g * 16 + lane, 1), :],
                sem,
            )
        return 0

    lax.fori_loop(0, _B_PER_W // 16, body, 0)

    def drain(j, _):
        pltpu.make_async_copy(
            table_hbm.at[pl.ds(0, 1), :],
            out_hbm.at[pl.ds(base, 1), :],
            sem,
        ).wait()
        return 0

    lax.fori_loop(0, _B_PER_W, drain, 0, unroll=8)


def kernel(t, embed_weight):
    return _gather_kernel(embed_weight, t.astype(jnp.int32))


# speedup vs baseline: 5.7614x; 5.7614x over previous
import functools

import jax
import jax.numpy as jnp
from jax import lax
from jax.experimental import pallas as pl
from jax.experimental.pallas import tpu as pltpu
from jax.experimental.pallas import tpu_sc as plsc

_MAX_T = 100000
_EMB_DIM = 16
_BATCH = 16384

_NC = 2
_NS = 16
_NW = _NC * _NS
_B_PER_W = _BATCH // _NW

_mesh = plsc.VectorSubcoreMesh(core_axis_name="c", subcore_axis_name="s")


@functools.partial(
    pl.kernel,
    mesh=_mesh,
    out_type=jax.ShapeDtypeStruct((_EMB_DIM * _BATCH,), jnp.float32),
    scratch_types=[
        pltpu.VMEM((_B_PER_W,), jnp.int32),
        pltpu.VMEM((_B_PER_W,), jnp.int32),
        pltpu.VMEM((_B_PER_W,), jnp.float32),
        pltpu.SemaphoreType.DMA,
    ],
    compiler_params=pltpu.CompilerParams(use_tc_tiling_on_sc=False),
)
def _gather_kernel(flat_hbm, idx_hbm, out_hbm, idx_v, idx_d, val_v, sem):
    cid = lax.axis_index("c")
    sid = lax.axis_index("s")
    wid = sid * _NC + cid
    base = wid * _B_PER_W

    pltpu.sync_copy(idx_hbm.at[pl.ds(base, _B_PER_W)], idx_v)

    def feature(d, _):
        def shift(g, _):
            idx_d[pl.ds(g * 16, 16)] = idx_v[pl.ds(g * 16, 16)] + d * _MAX_T
            return 0

        lax.fori_loop(0, _B_PER_W // 16, shift, 0)
        pltpu.async_copy(flat_hbm.at[idx_d], val_v, sem).wait()
        pltpu.sync_copy(val_v, out_hbm.at[pl.ds(d * _BATCH + base, _B_PER_W)])
        return 0

    lax.fori_loop(0, _EMB_DIM, feature, 0)


def kernel(t, embed_weight):
    flat = embed_weight.T.reshape(_EMB_DIM * _MAX_T)
    out_t = _gather_kernel(flat, t.astype(jnp.int32))
    return out_t.reshape(_EMB_DIM, _BATCH).T


# speedup vs baseline: 7.3047x; 1.2679x over previous
import functools

import jax
import jax.numpy as jnp
from jax import lax
from jax.experimental import pallas as pl
from jax.experimental.pallas import tpu as pltpu
from jax.experimental.pallas import tpu_sc as plsc

_MAX_T = 100000
_EMB_DIM = 16
_BATCH = 16384

_NC = 2
_NS = 16
_NW = _NC * _NS
_B_PER_W = _BATCH // _NW

_mesh = plsc.VectorSubcoreMesh(core_axis_name="c", subcore_axis_name="s")


@functools.partial(
    pl.kernel,
    mesh=_mesh,
    out_type=jax.ShapeDtypeStruct((_EMB_DIM * _BATCH,), jnp.float32),
    scratch_types=[
        pltpu.VMEM((_B_PER_W,), jnp.int32),
        pltpu.VMEM((_EMB_DIM * _B_PER_W,), jnp.int32),
        pltpu.VMEM((_EMB_DIM * _B_PER_W,), jnp.float32),
        pltpu.SemaphoreType.DMA,
        pltpu.SemaphoreType.DMA,
    ],
    compiler_params=pltpu.CompilerParams(use_tc_tiling_on_sc=False),
)
def _gather_kernel(flat_hbm, idx_hbm, out_hbm, idx_v, idx_all, val_all, sem, wsem):
    cid = lax.axis_index("c")
    sid = lax.axis_index("s")
    wid = sid * _NC + cid
    base = wid * _B_PER_W

    pltpu.sync_copy(idx_hbm.at[pl.ds(base, _B_PER_W)], idx_v)

    def shift(i, _):
        d = i // (_B_PER_W // 16)
        g = i % (_B_PER_W // 16)
        idx_all[pl.ds(i * 16, 16)] = idx_v[pl.ds(g * 16, 16)] + d * _MAX_T
        return 0

    lax.fori_loop(0, _EMB_DIM * _B_PER_W // 16, shift, 0, unroll=8)

    pltpu.async_copy(flat_hbm.at[idx_all], val_all, sem).wait()

    copies = [
        pltpu.async_copy(
            val_all.at[pl.ds(d * _B_PER_W, _B_PER_W)],
            out_hbm.at[pl.ds(d * _BATCH + base, _B_PER_W)],
            wsem,
        )
        for d in range(_EMB_DIM)
    ]
    for c in copies:
        c.wait()


def kernel(t, embed_weight):
    flat = embed_weight.T.reshape(_EMB_DIM * _MAX_T)
    out_t = _gather_kernel(flat, t.astype(jnp.int32))
    return out_t.reshape(_EMB_DIM, _BATCH).T


# speedup vs baseline: 7.7519x; 1.0612x over previous
import functools

import jax
import jax.numpy as jnp
from jax import lax
from jax.experimental import pallas as pl
from jax.experimental.pallas import tpu as pltpu
from jax.experimental.pallas import tpu_sc as plsc

_MAX_T = 100000
_EMB_DIM = 16
_BATCH = 16384

_NC = 2
_NS = 16
_NW = _NC * _NS
_B_PER_W = _BATCH // _NW

_mesh = plsc.VectorSubcoreMesh(core_axis_name="c", subcore_axis_name="s")


@functools.partial(
    pl.kernel,
    mesh=_mesh,
    out_type=jax.ShapeDtypeStruct((_EMB_DIM * _BATCH,), jnp.float32),
    scratch_types=[
        pltpu.VMEM((_B_PER_W,), jnp.int32),
        [pltpu.VMEM((_B_PER_W,), jnp.float32) for _ in range(_EMB_DIM)],
        [pltpu.SemaphoreType.DMA for _ in range(_EMB_DIM)],
        pltpu.SemaphoreType.DMA,
    ],
    compiler_params=pltpu.CompilerParams(use_tc_tiling_on_sc=False),
)
def _gather_kernel(flat_hbm, idx_hbm, out_hbm, idx_v, vals, sems, wsem):
    cid = lax.axis_index("c")
    sid = lax.axis_index("s")
    wid = sid * _NC + cid
    base = wid * _B_PER_W

    pltpu.sync_copy(idx_hbm.at[pl.ds(base, _B_PER_W)], idx_v)

    gathers = [
        pltpu.async_copy(
            flat_hbm.at[pl.ds(d * _MAX_T, _MAX_T)].at[idx_v],
            vals[d],
            sems[d],
        )
        for d in range(_EMB_DIM)
    ]
    writes = []
    for d in range(_EMB_DIM):
        gathers[d].wait()
        writes.append(
            pltpu.async_copy(
                vals[d], out_hbm.at[pl.ds(d * _BATCH + base, _B_PER_W)], wsem
            )
        )
    for w in writes:
        w.wait()


def kernel(t, embed_weight):
    flat = embed_weight.T.reshape(_EMB_DIM * _MAX_T)
    out_t = _gather_kernel(flat, t.astype(jnp.int32))
    return out_t.reshape(_EMB_DIM, _BATCH).T
